# baseline (device time: 237515 ns/iter reference)
import os

import jax

_CACHE_DIR = os.path.join(os.path.dirname(os.path.abspath(__file__)), ".jax_cache")
jax.config.update("jax_compilation_cache_dir", _CACHE_DIR)
jax.config.update("jax_persistent_cache_min_compile_time_secs", 0.0)
jax.config.update("jax_persistent_cache_min_entry_size_bytes", 0)

import jax.numpy as jnp
from jax import lax
from jax.experimental import pallas as pl
from jax.experimental.pallas import tpu as pltpu

N_DEV = 8
SQ = 2048
SKV_PER = 2048
HQ = 8
DH = 128
DM = 1024
SCALE = 0.08838834764831843
QB = 512
N_QB = SQ // QB


def _partial_body(x_ref, wq_ref, k_ref, v_ref, acc_ref, l_ref):
    my = lax.axis_index("i")

    acc_ref[...] = jnp.zeros((SQ, DM), jnp.float32)
    l_ref[...] = jnp.zeros((SQ, HQ), jnp.float32)

    def attend(qb, h, klo, kw, extra=None):
        qh = jnp.dot(
            x_ref[0, qb * QB:(qb + 1) * QB, :],
            wq_ref[:, h * DH:(h + 1) * DH],
            preferred_element_type=jnp.float32,
        )
        windows = [(klo, kw)] + ([extra] if extra else [])
        acc_p, l_p = None, None
        for lo, w in windows:
            qi = lax.broadcasted_iota(jnp.int32, (QB, w), 0) + qb * QB
            kig = lax.broadcasted_iota(jnp.int32, (QB, w), 1) + lo + my * SKV_PER
            mask = (jnp.abs(qi - kig) <= 128) | (kig < 32) | (qi < 32)
            kh = k_ref[0, lo:lo + w, h, :]
            vh = v_ref[0, lo:lo + w, h, :]
            s = lax.dot_general(
                qh, kh, (((1,), (1,)), ((), ())),
                preferred_element_type=jnp.float32,
            ) * SCALE
            p = jnp.where(mask, jnp.exp(s), 0.0)
            a = jnp.dot(p, vh, preferred_element_type=jnp.float32)
            ll = jnp.sum(p, axis=1, keepdims=True)
            acc_p = a if acc_p is None else acc_p + a
            l_p = ll if l_p is None else l_p + ll
        acc_ref[qb * QB:(qb + 1) * QB, h * DH:(h + 1) * DH] = acc_p
        l_ref[qb * QB:(qb + 1) * QB, h:h + 1] = l_p

    for h in range(HQ):
        attend(0, h, 0, SKV_PER)

    @pl.when(my == 0)
    def _():
        for qb in range(1, N_QB):
            klo = qb * QB - 128
            kw = min(768, SKV_PER - klo)
            for h in range(HQ):
                attend(qb, h, klo, kw, extra=(0, 128))

    @pl.when(my == 1)
    def _():
        for h in range(HQ):
            attend(N_QB - 1, h, 0, 128)


XMASKS = (1, 3, 4)
BCAST_LO = 128
BCAST_HI = 1920
CHUNK_ROWS = 256
N_CHUNK = (BCAST_HI - BCAST_LO) // CHUNK_ROWS


def _ring_body(acc_ref, l_ref, wo_ref, out_ref,
               xa, xl, ra, rl, bc, ctx_ref,
               xa_send, xa_recv, xl_send, xl_recv, ch_send, ch_recv):
    my = lax.axis_index("i")
    right = (my + 1) % N_DEV
    left = (my + N_DEV - 1) % N_DEV

    barrier_sem = pltpu.get_barrier_semaphore()
    for nbr in (my ^ 1, my ^ 3, my ^ 4, right, left):
        pl.semaphore_signal(
            barrier_sem, inc=1,
            device_id=(nbr,), device_id_type=pl.DeviceIdType.MESH,
        )
    pl.semaphore_wait(barrier_sem, 5)

    xa[0:128] = acc_ref[0:128]
    xa[128:256] = acc_ref[BCAST_HI:SQ]
    xl[...] = l_ref[...]

    def chunk_copy(d, c, tgt):
        rows = pl.ds(c * CHUNK_ROWS, CHUNK_ROWS)
        return pltpu.make_async_remote_copy(
            src_ref=bc.at[rows], dst_ref=bc.at[rows],
            send_sem=ch_send.at[d, c], recv_sem=ch_recv.at[d, c],
            device_id=(tgt,), device_id_type=pl.DeviceIdType.MESH,
        )

    @pl.when(my == 0)
    def _():
        bc[...] = acc_ref[BCAST_LO:BCAST_HI, :].astype(jnp.bfloat16)
        for c in range(N_CHUNK):
            chunk_copy(0, c, right).start()
            chunk_copy(1, c, left).start()

    for r, _m in enumerate(XMASKS):
        partner = my ^ _m
        ea = pltpu.make_async_remote_copy(
            src_ref=xa, dst_ref=ra.at[r],
            send_sem=xa_send.at[r], recv_sem=xa_recv.at[r],
            device_id=(partner,), device_id_type=pl.DeviceIdType.MESH,
        )
        el = pltpu.make_async_remote_copy(
            src_ref=xl, dst_ref=rl.at[r],
            send_sem=xl_send.at[r], recv_sem=xl_recv.at[r],
            device_id=(partner,), device_id_type=pl.DeviceIdType.MESH,
        )
        ea.start()
        el.start()
        ea.wait()
        el.wait()
        xa[...] = xa[...] + ra[r]
        xl[...] = xl[...] + rl[r]

    for c in range(N_CHUNK):
        dr = chunk_copy(0, c, right)
        dl = chunk_copy(1, c, left)

        @pl.when((my >= 1) & (my <= 4))
        def _():
            dr.wait_recv()

        @pl.when((my >= 1) & (my <= 3))
        def _():
            dr.start()

        @pl.when(my >= 5)
        def _():
            dl.wait_recv()

        @pl.when(my >= 6)
        def _():
            dl.start()

    for h in range(HQ):
        hc = slice(h * DH, (h + 1) * DH)
        ctx_ref[0:128, hc] = xa[0:128, hc] / xl[0:128, h:h + 1]
        ctx_ref[BCAST_HI:SQ, hc] = xa[128:256, hc] / xl[BCAST_HI:SQ, h:h + 1]
        ctx_ref[BCAST_LO:BCAST_HI, hc] = (
            bc[:, hc].astype(jnp.float32) / xl[BCAST_LO:BCAST_HI, h:h + 1]
        )
    out_ref[0] = jnp.dot(ctx_ref[...], wo_ref[...],
                         preferred_element_type=jnp.float32)

    @pl.when(my <= 3)
    def _():
        for c in range(N_CHUNK):
            chunk_copy(0, c, right).wait_send()

    @pl.when((my == 0) | (my >= 6))
    def _():
        for c in range(N_CHUNK):
            chunk_copy(1, c, left).wait_send()


def kernel(x, Wq, K_ext, V_ext, Wo):
    f32 = jnp.float32

    acc, l = pl.pallas_call(
        _partial_body,
        out_shape=(
            jax.ShapeDtypeStruct((SQ, DM), f32),
            jax.ShapeDtypeStruct((SQ, HQ), f32),
        ),
        in_specs=[pl.BlockSpec(memory_space=pltpu.VMEM)] * 4,
        out_specs=(
            pl.BlockSpec(memory_space=pltpu.VMEM),
            pl.BlockSpec(memory_space=pltpu.VMEM),
        ),
    )(x, Wq, K_ext, V_ext)

    out = pl.pallas_call(
        _ring_body,
        out_shape=jax.ShapeDtypeStruct((1, SQ, DM), f32),
        in_specs=[pl.BlockSpec(memory_space=pltpu.VMEM)] * 3,
        out_specs=pl.BlockSpec(memory_space=pltpu.VMEM),
        scratch_shapes=[
            pltpu.VMEM((256, DM), f32),
            pltpu.VMEM((SQ, HQ), f32),
            pltpu.VMEM((3, 256, DM), f32),
            pltpu.VMEM((3, SQ, HQ), f32),
            pltpu.VMEM((BCAST_HI - BCAST_LO, DM), jnp.bfloat16),
            pltpu.VMEM((SQ, DM), f32),
            pltpu.SemaphoreType.DMA((3,)),
            pltpu.SemaphoreType.DMA((3,)),
            pltpu.SemaphoreType.DMA((3,)),
            pltpu.SemaphoreType.DMA((3,)),
            pltpu.SemaphoreType.DMA((2, N_CHUNK)),
            pltpu.SemaphoreType.DMA((2, N_CHUNK)),
        ],
        compiler_params=pltpu.CompilerParams(
            collective_id=0, has_side_effects=True
        ),
    )(acc, l, Wo)

    return out


# device time: 208924 ns/iter; 1.1368x vs baseline; 1.1368x over previous
import os

import jax

_CACHE_DIR = os.path.join(os.path.dirname(os.path.abspath(__file__)), ".jax_cache")
jax.config.update("jax_compilation_cache_dir", _CACHE_DIR)
jax.config.update("jax_persistent_cache_min_compile_time_secs", 0.0)
jax.config.update("jax_persistent_cache_min_entry_size_bytes", 0)

import jax.numpy as jnp
from jax import lax
from jax.experimental import pallas as pl
from jax.experimental.pallas import tpu as pltpu

N_DEV = 8
SQ = 2048
SKV_PER = 2048
HQ = 8
DH = 128
DM = 1024
SCALE = 0.08838834764831843
QB = 512
N_QB = SQ // QB


def _partial_body(x_ref, wq_ref, k_ref, v_ref, acc_ref, l_ref):
    my = lax.axis_index("i")

    acc_ref[...] = jnp.zeros((SQ, DM), jnp.float32)
    l_ref[...] = jnp.zeros((SQ, HQ), jnp.float32)

    def attend(qb, h, klo, kw, extra=None):
        qh = jnp.dot(
            x_ref[0, qb * QB:(qb + 1) * QB, :],
            wq_ref[:, h * DH:(h + 1) * DH],
            preferred_element_type=jnp.float32,
        )
        windows = [(klo, kw)] + ([extra] if extra else [])
        acc_p, l_p = None, None
        for lo, w in windows:
            qi = lax.broadcasted_iota(jnp.int32, (QB, w), 0) + qb * QB
            kig = lax.broadcasted_iota(jnp.int32, (QB, w), 1) + lo + my * SKV_PER
            mask = (jnp.abs(qi - kig) <= 128) | (kig < 32) | (qi < 32)
            kh = k_ref[0, lo:lo + w, h, :]
            vh = v_ref[0, lo:lo + w, h, :]
            s = lax.dot_general(
                qh, kh, (((1,), (1,)), ((), ())),
                preferred_element_type=jnp.float32,
            ) * SCALE
            p = jnp.where(mask, jnp.exp(s), 0.0)
            a = jnp.dot(p, vh, preferred_element_type=jnp.float32)
            ll = jnp.sum(p, axis=1, keepdims=True)
            acc_p = a if acc_p is None else acc_p + a
            l_p = ll if l_p is None else l_p + ll
        acc_ref[qb * QB:(qb + 1) * QB, h * DH:(h + 1) * DH] = acc_p
        l_ref[qb * QB:(qb + 1) * QB, h:h + 1] = l_p

    for h in range(HQ):
        attend(0, h, 0, SKV_PER)

    @pl.when(my == 0)
    def _():
        for qb in range(1, N_QB):
            klo = qb * QB - 128
            kw = min(768, SKV_PER - klo)
            for h in range(HQ):
                attend(qb, h, klo, kw, extra=(0, 128))

    @pl.when(my == 1)
    def _():
        for h in range(HQ):
            attend(N_QB - 1, h, 0, 128)


XMASKS = (1, 3, 4)
BCAST_LO = 128
BCAST_HI = 1920
CHUNK_ROWS = 256
N_CHUNK = (BCAST_HI - BCAST_LO) // CHUNK_ROWS


def _ring_body(acc_ref, l_ref, wo_ref, out_ref,
               xa, xl, ra, rl, bc, ctx_ref,
               xa_send, xa_recv, xl_send, xl_recv, ch_send, ch_recv):
    my = lax.axis_index("i")
    right = (my + 1) % N_DEV
    left = (my + N_DEV - 1) % N_DEV

    barrier_sem = pltpu.get_barrier_semaphore()
    for nbr in (my ^ 1, my ^ 3, my ^ 4, right, left):
        pl.semaphore_signal(
            barrier_sem, inc=1,
            device_id=(nbr,), device_id_type=pl.DeviceIdType.MESH,
        )
    pl.semaphore_wait(barrier_sem, 5)

    xa[0:128] = acc_ref[0:128]
    xa[128:256] = acc_ref[BCAST_HI:SQ]
    xl[0:128] = l_ref[0:128]
    xl[128:256] = l_ref[BCAST_HI:SQ]

    def chunk_copy(d, c, tgt):
        rows = pl.ds(c * CHUNK_ROWS, CHUNK_ROWS)
        return pltpu.make_async_remote_copy(
            src_ref=bc.at[rows], dst_ref=bc.at[rows],
            send_sem=ch_send.at[d, c], recv_sem=ch_recv.at[d, c],
            device_id=(tgt,), device_id_type=pl.DeviceIdType.MESH,
        )

    @pl.when(my == 0)
    def _():
        for h in range(HQ):
            hc = slice(h * DH, (h + 1) * DH)
            bc[:, hc] = (
                acc_ref[BCAST_LO:BCAST_HI, hc]
                / l_ref[BCAST_LO:BCAST_HI, h:h + 1]
            ).astype(jnp.bfloat16)
        for c in range(N_CHUNK):
            chunk_copy(0, c, right).start()
            chunk_copy(1, c, left).start()

    for r, _m in enumerate(XMASKS):
        partner = my ^ _m
        ea = pltpu.make_async_remote_copy(
            src_ref=xa, dst_ref=ra.at[r],
            send_sem=xa_send.at[r], recv_sem=xa_recv.at[r],
            device_id=(partner,), device_id_type=pl.DeviceIdType.MESH,
        )
        el = pltpu.make_async_remote_copy(
            src_ref=xl, dst_ref=rl.at[r],
            send_sem=xl_send.at[r], recv_sem=xl_recv.at[r],
            device_id=(partner,), device_id_type=pl.DeviceIdType.MESH,
        )
        ea.start()
        el.start()
        ea.wait()
        el.wait()
        xa[...] = xa[...] + ra[r]
        xl[...] = xl[...] + rl[r]

    for c in range(N_CHUNK):
        dr = chunk_copy(0, c, right)
        dl = chunk_copy(1, c, left)

        @pl.when((my >= 1) & (my <= 4))
        def _():
            dr.wait_recv()

        @pl.when((my >= 1) & (my <= 3))
        def _():
            dr.start()

        @pl.when(my >= 5)
        def _():
            dl.wait_recv()

        @pl.when(my >= 6)
        def _():
            dl.start()

    for h in range(HQ):
        hc = slice(h * DH, (h + 1) * DH)
        ctx_ref[0:128, hc] = xa[0:128, hc] / xl[0:128, h:h + 1]
        ctx_ref[BCAST_HI:SQ, hc] = xa[128:256, hc] / xl[128:256, h:h + 1]
    ctx_ref[BCAST_LO:BCAST_HI, :] = bc[...].astype(jnp.float32)
    out_ref[0] = jnp.dot(ctx_ref[...], wo_ref[...],
                         preferred_element_type=jnp.float32)

    @pl.when(my <= 3)
    def _():
        for c in range(N_CHUNK):
            chunk_copy(0, c, right).wait_send()

    @pl.when((my == 0) | (my >= 6))
    def _():
        for c in range(N_CHUNK):
            chunk_copy(1, c, left).wait_send()


def kernel(x, Wq, K_ext, V_ext, Wo):
    f32 = jnp.float32

    acc, l = pl.pallas_call(
        _partial_body,
        out_shape=(
            jax.ShapeDtypeStruct((SQ, DM), f32),
            jax.ShapeDtypeStruct((SQ, HQ), f32),
        ),
        in_specs=[pl.BlockSpec(memory_space=pltpu.VMEM)] * 4,
        out_specs=(
            pl.BlockSpec(memory_space=pltpu.VMEM),
            pl.BlockSpec(memory_space=pltpu.VMEM),
        ),
    )(x, Wq, K_ext, V_ext)

    out = pl.pallas_call(
        _ring_body,
        out_shape=jax.ShapeDtypeStruct((1, SQ, DM), f32),
        in_specs=[pl.BlockSpec(memory_space=pltpu.VMEM)] * 3,
        out_specs=pl.BlockSpec(memory_space=pltpu.VMEM),
        scratch_shapes=[
            pltpu.VMEM((256, DM), f32),
            pltpu.VMEM((256, HQ), f32),
            pltpu.VMEM((3, 256, DM), f32),
            pltpu.VMEM((3, 256, HQ), f32),
            pltpu.VMEM((BCAST_HI - BCAST_LO, DM), jnp.bfloat16),
            pltpu.VMEM((SQ, DM), f32),
            pltpu.SemaphoreType.DMA((3,)),
            pltpu.SemaphoreType.DMA((3,)),
            pltpu.SemaphoreType.DMA((3,)),
            pltpu.SemaphoreType.DMA((3,)),
            pltpu.SemaphoreType.DMA((2, N_CHUNK)),
            pltpu.SemaphoreType.DMA((2, N_CHUNK)),
        ],
        compiler_params=pltpu.CompilerParams(
            collective_id=0, has_side_effects=True
        ),
    )(acc, l, Wo)

    return out
